# Initial kernel scaffold; baseline (speedup 1.0000x reference)
import jax
import jax.numpy as jnp
from jax.experimental import pallas as pl


def kernel(x, edge_index, graph_ids, gin_w1_0, gin_b1_0, gin_w2_0, gin_b2_0, gin_w1, gin_b1, gin_w2, gin_b2, pred_w0, pred_b0, pred_w, pred_b, mlp_w1, mlp_b1, mlp_w2, mlp_b2, mean_w, mean_b, std_w, std_b):
    raise NotImplementedError("write your pallas kernel here")


# speedup vs baseline: 3.2505x; 3.2505x over previous
import functools

import jax
import jax.numpy as jnp
from jax import lax
from jax.experimental import pallas as pl
from jax.experimental.pallas import tpu as pltpu
from jax.experimental.pallas import tpu_sc as plsc

N = 50000
E = 800000
H = 128
OUT = 256
BOT = 128
LAT = 64
BG = 50
BGP = 64

N_PAD = 50176
N_ACC = 50304
E_PAD = 819200
KI = 4
BLK = 512


def _sc_segsum_pair(x16, src2d, dst2d, zeros16):
    mesh = plsc.VectorSubcoreMesh(core_axis_name="c", subcore_axis_name="s")
    F = x16.shape[1]
    rows_per_worker = E_PAD // 32 // 128
    n_macros = rows_per_worker // KI
    zrows = N_ACC // 16
    orows = N_PAD // 16

    @functools.partial(
        pl.kernel,
        out_type=jax.ShapeDtypeStruct((2, N_PAD, F), jnp.float32),
        mesh=mesh,
        compiler_params=pltpu.CompilerParams(use_tc_tiling_on_sc=False),
        scratch_types=[
            pltpu.VMEM((KI, 128), jnp.int32),
            pltpu.VMEM((KI, 128), jnp.int32),
            pltpu.VMEM((KI * 128, F), jnp.float32),
            pltpu.VMEM_SHARED((N_ACC, F), jnp.float32),
            pltpu.SemaphoreType.DMA,
            pltpu.SemaphoreType.DMA,
        ],
    )
    def k(x_hbm, src_hbm, dst_hbm, z_hbm, out_hbm, idx_s, idx_d, rows, acc,
          sem_g, sem_s):
        cid = lax.axis_index("c")
        sid = lax.axis_index("s")
        wid = sid * 2 + cid

        pltpu.sync_copy(z_hbm.at[pl.ds(sid * zrows, zrows)],
                        acc.at[pl.ds(sid * zrows, zrows)])
        plsc.subcore_barrier()

        row0 = wid * rows_per_worker

        @pl.loop(0, n_macros)
        def _(m):
            rbase = row0 + m * KI
            pltpu.sync_copy(src_hbm.at[pl.ds(rbase, KI)], idx_s)
            pltpu.sync_copy(dst_hbm.at[pl.ds(rbase, KI)], idx_d)
            gs = [pltpu.async_copy(x_hbm.at[idx_s.at[j]],
                                   rows.at[pl.ds(j * 128, 128)], sem_g)
                  for j in range(KI)]
            for g in gs:
                g.wait()
            for j in range(KI):
                pltpu.sync_copy(rows.at[pl.ds(j * 128, 128)],
                                acc.at[idx_d.at[j]], add=True)

        plsc.subcore_barrier()
        pltpu.sync_copy(acc.at[pl.ds(sid * orows, orows)],
                        out_hbm.at[cid].at[pl.ds(sid * orows, orows)])

    return k(x16, src2d, dst2d, zeros16)


def _sc_segsum_one_round(hA, hB, src2d, dst2d, zeros32):
    mesh = plsc.VectorSubcoreMesh(core_axis_name="c", subcore_axis_name="s")
    F = hA.shape[1]
    rows_per_sub = E_PAD // 16 // 128
    n_macros = rows_per_sub // KI
    zrows = N_ACC // 16
    orows = N_PAD // 16
    ot = jax.ShapeDtypeStruct((N_PAD, F), jnp.float32)

    @functools.partial(
        pl.kernel,
        out_type=(ot, ot),
        mesh=mesh,
        compiler_params=pltpu.CompilerParams(use_tc_tiling_on_sc=False),
        scratch_types=[
            pltpu.VMEM((KI, 128), jnp.int32),
            pltpu.VMEM((KI, 128), jnp.int32),
            pltpu.VMEM((KI * 128, F), jnp.float32),
            pltpu.VMEM_SHARED((N_ACC, F), jnp.float32),
            pltpu.SemaphoreType.DMA,
            pltpu.SemaphoreType.DMA,
        ],
    )
    def k(hA_hbm, hB_hbm, src_hbm, dst_hbm, z_hbm, oA_hbm, oB_hbm,
          idx_s, idx_d, rows, acc, sem_g, sem_s):
        cid = lax.axis_index("c")
        sid = lax.axis_index("s")
        row0 = sid * rows_per_sub

        def do_chunk(h_hbm, o_hbm):
            pltpu.sync_copy(z_hbm.at[pl.ds(sid * zrows, zrows)],
                            acc.at[pl.ds(sid * zrows, zrows)])
            plsc.subcore_barrier()

            @pl.loop(0, n_macros)
            def _(m):
                rbase = row0 + m * KI
                pltpu.sync_copy(src_hbm.at[pl.ds(rbase, KI)], idx_s)
                pltpu.sync_copy(dst_hbm.at[pl.ds(rbase, KI)], idx_d)
                gs = [pltpu.async_copy(h_hbm.at[idx_s.at[j]],
                                       rows.at[pl.ds(j * 128, 128)], sem_g)
                      for j in range(KI)]
                for g in gs:
                    g.wait()
                for j in range(KI):
                    pltpu.sync_copy(rows.at[pl.ds(j * 128, 128)],
                                    acc.at[idx_d.at[j]], add=True)

            plsc.subcore_barrier()
            pltpu.sync_copy(acc.at[pl.ds(sid * orows, orows)],
                            o_hbm.at[pl.ds(sid * orows, orows)])

        @pl.when(cid == 0)
        def _():
            do_chunk(hA_hbm, oA_hbm)

        @pl.when(cid == 1)
        def _():
            do_chunk(hB_hbm, oB_hbm)

    return k(hA, hB, src2d, dst2d, zeros32)


def _iota_cols(shape):
    return lax.broadcasted_iota(jnp.int32, shape, dimension=1)


def _tc_layer0(p0, x16, gids2d, w1p, b1, w2, b2):
    grid = N_PAD // BLK
    oc = jax.ShapeDtypeStruct((N_PAD, 32), jnp.float32)

    def body(p_ref, x_ref, g_ref, w1_ref, b1_ref, w2_ref, b2_ref,
             o0, o1, o2, o3, s0_ref, s1_ref):
        i = pl.program_id(0)
        a = p_ref[0] + p_ref[1] + x_ref[...]
        t = jnp.maximum(jnp.dot(a, w1_ref[...],
                                preferred_element_type=jnp.float32)
                        + b1_ref[...], 0.0)
        h = jnp.maximum(jnp.dot(t, w2_ref[...],
                                preferred_element_type=jnp.float32)
                        + b2_ref[...], 0.0)
        outs = (o0, o1, o2, o3)
        for c in range(4):
            outs[c][...] = h[:, c * 32:(c + 1) * 32]
        ind = (g_ref[...] == _iota_cols((BLK, BGP))).astype(jnp.float32)
        c0 = lax.dot_general(ind, x_ref[...], (((0,), (0,)), ((), ())),
                             preferred_element_type=jnp.float32)
        c1 = lax.dot_general(ind, h, (((0,), (0,)), ((), ())),
                             preferred_element_type=jnp.float32)

        @pl.when(i == 0)
        def _():
            s0_ref[...] = c0
            s1_ref[...] = c1

        @pl.when(i > 0)
        def _():
            s0_ref[...] += c0
            s1_ref[...] += c1

    return pl.pallas_call(
        body,
        grid=(grid,),
        in_specs=[
            pl.BlockSpec((2, BLK, 16), lambda i: (0, i, 0)),
            pl.BlockSpec((BLK, 16), lambda i: (i, 0)),
            pl.BlockSpec((BLK, 1), lambda i: (i, 0)),
            pl.BlockSpec((16, H), lambda i: (0, 0)),
            pl.BlockSpec((1, H), lambda i: (0, 0)),
            pl.BlockSpec((H, H), lambda i: (0, 0)),
            pl.BlockSpec((1, H), lambda i: (0, 0)),
        ],
        out_specs=[
            pl.BlockSpec((BLK, 32), lambda i: (i, 0)),
            pl.BlockSpec((BLK, 32), lambda i: (i, 0)),
            pl.BlockSpec((BLK, 32), lambda i: (i, 0)),
            pl.BlockSpec((BLK, 32), lambda i: (i, 0)),
            pl.BlockSpec((BGP, 16), lambda i: (0, 0)),
            pl.BlockSpec((BGP, H), lambda i: (0, 0)),
        ],
        out_shape=(oc, oc, oc, oc,
                   jax.ShapeDtypeStruct((BGP, 16), jnp.float32),
                   jax.ShapeDtypeStruct((BGP, H), jnp.float32)),
    )(p0, x16, gids2d, w1p, b1, w2, b2)


def _tc_layer(aggs, hs, gids2d, w1, b1, w2, b2, emit_h):
    grid = N_PAD // BLK
    oc = jax.ShapeDtypeStruct((N_PAD, 32), jnp.float32)

    def body(*refs):
        (a0, a1, a2, a3, h0, h1, h2, h3, g_ref, w1_ref, b1_ref, w2_ref,
         b2_ref) = refs[:13]
        outs = refs[13:]
        i = pl.program_id(0)
        arefs = (a0, a1, a2, a3)
        hrefs = (h0, h1, h2, h3)
        a = jnp.concatenate(
            [arefs[c][...] + hrefs[c][...] for c in range(4)], axis=1)
        t = jnp.maximum(jnp.dot(a, w1_ref[...],
                                preferred_element_type=jnp.float32)
                        + b1_ref[...], 0.0)
        h = jnp.maximum(jnp.dot(t, w2_ref[...],
                                preferred_element_type=jnp.float32)
                        + b2_ref[...], 0.0)
        if emit_h:
            for c in range(4):
                outs[c][...] = h[:, c * 32:(c + 1) * 32]
        s_ref = outs[-1]
        ind = (g_ref[...] == _iota_cols((BLK, BGP))).astype(jnp.float32)
        contrib = lax.dot_general(ind, h, (((0,), (0,)), ((), ())),
                                  preferred_element_type=jnp.float32)

        @pl.when(i == 0)
        def _():
            s_ref[...] = contrib

        @pl.when(i > 0)
        def _():
            s_ref[...] += contrib

    chunk_spec = pl.BlockSpec((BLK, 32), lambda i: (i, 0))
    in_specs = ([chunk_spec] * 4 + [chunk_spec] * 4 +
                [pl.BlockSpec((BLK, 1), lambda i: (i, 0)),
                 pl.BlockSpec((H, H), lambda i: (0, 0)),
                 pl.BlockSpec((1, H), lambda i: (0, 0)),
                 pl.BlockSpec((H, H), lambda i: (0, 0)),
                 pl.BlockSpec((1, H), lambda i: (0, 0))])
    s_shape = jax.ShapeDtypeStruct((BGP, H), jnp.float32)
    s_spec = pl.BlockSpec((BGP, H), lambda i: (0, 0))
    if emit_h:
        out_specs = [chunk_spec] * 4 + [s_spec]
        out_shape = (oc, oc, oc, oc, s_shape)
    else:
        out_specs = [s_spec]
        out_shape = (s_shape,)

    return pl.pallas_call(
        body,
        grid=(grid,),
        in_specs=in_specs,
        out_specs=out_specs,
        out_shape=out_shape,
    )(*aggs, *hs, gids2d, w1, b1, w2, b2)


def _tc_head(s0, s1, s2, s3, s4, pw0p, pb0, pred_w, pred_b,
             mlp_w1, mlp_b1, mlp_w2, mlp_b2, mean_w, mean_b, std_w, std_b):
    def body(s0_ref, s1_ref, s2_ref, s3_ref, s4_ref, pw0_ref, pb0_ref,
             pw_ref, pb_ref, w1_ref, b1_ref, w2_ref, b2_ref,
             mw_ref, mb_ref, sw_ref, sb_ref, mean_ref, std_ref):
        dot = functools.partial(jnp.dot, preferred_element_type=jnp.float32)
        score = dot(s0_ref[...], pw0_ref[...]) + pb0_ref[...]
        srefs = (s1_ref, s2_ref, s3_ref, s4_ref)
        for i in range(4):
            score = score + dot(srefs[i][...], pw_ref[i])
        score = score + jnp.sum(pb_ref[...], axis=0, keepdims=True)
        feat = jnp.maximum(dot(score, w1_ref[...]) + b1_ref[...], 0.0)
        feat = jnp.maximum(dot(feat, w2_ref[...]) + b2_ref[...], 0.0)
        mean_ref[...] = dot(feat, mw_ref[...]) + mb_ref[...]
        z = dot(feat, sw_ref[...]) + sb_ref[...]
        std_ref[...] = jnp.maximum(z, 0.0) + jnp.log1p(jnp.exp(-jnp.abs(z)))

    full = lambda s: pl.BlockSpec(s, lambda: tuple(0 for _ in s))
    args = (s0, s1, s2, s3, s4, pw0p, pb0, pred_w, pred_b, mlp_w1, mlp_b1,
            mlp_w2, mlp_b2, mean_w, mean_b, std_w, std_b)
    return pl.pallas_call(
        body,
        in_specs=[full(a.shape) for a in args],
        out_specs=[full((BGP, LAT)), full((BGP, LAT))],
        out_shape=(jax.ShapeDtypeStruct((BGP, LAT), jnp.float32),
                   jax.ShapeDtypeStruct((BGP, LAT), jnp.float32)),
    )(*args)


def kernel(x, edge_index, graph_ids, gin_w1_0, gin_b1_0, gin_w2_0, gin_b2_0,
           gin_w1, gin_b1, gin_w2, gin_b2, pred_w0, pred_b0, pred_w, pred_b,
           mlp_w1, mlp_b1, mlp_w2, mlp_b2, mean_w, mean_b, std_w, std_b):
    src = edge_index[0]
    dst = edge_index[1]
    pad_e = E_PAD - E
    src2d = jnp.concatenate(
        [src, jnp.zeros((pad_e,), jnp.int32)]).reshape(E_PAD // 128, 128)
    dst2d = jnp.concatenate(
        [dst, jnp.full((pad_e,), N_PAD, jnp.int32)]).reshape(E_PAD // 128, 128)

    x16 = jnp.pad(x, ((0, N_PAD - N), (0, 12)))
    gids2d = jnp.pad(graph_ids, (0, N_PAD - N),
                     constant_values=BGP - 1).reshape(N_PAD, 1)
    zeros16 = jnp.zeros((N_ACC, 16), jnp.float32)
    zeros32 = jnp.zeros((N_ACC, 32), jnp.float32)

    w1_0p = jnp.pad(gin_w1_0, ((0, 12), (0, 0)))
    pw0p = jnp.pad(pred_w0, ((0, 12), (0, 0)))
    b1_0 = gin_b1_0.reshape(1, H)
    b2_0 = gin_b2_0.reshape(1, H)
    pb0 = pred_b0.reshape(1, OUT)

    def sc_big(h0, h1, h2, h3):
        a0, a2 = _sc_segsum_one_round(h0, h2, src2d, dst2d, zeros32)
        a1, a3 = _sc_segsum_one_round(h1, h3, src2d, dst2d, zeros32)
        return (a0, a1, a2, a3)

    p0 = _sc_segsum_pair(x16, src2d, dst2d, zeros16)
    h1c = _tc_layer0(p0, x16, gids2d, w1_0p, b1_0, gin_w2_0, b2_0)
    hc = h1c[:4]
    s0, s1 = h1c[4], h1c[5]

    ss = [s1]
    for i in range(3):
        agg = sc_big(*hc)
        res = _tc_layer(agg, hc, gids2d, gin_w1[i], gin_b1[i].reshape(1, H),
                        gin_w2[i], gin_b2[i].reshape(1, H), emit_h=(i < 2))
        if i < 2:
            hc = res[:4]
            ss.append(res[4])
        else:
            ss.append(res[0])

    mean_p, std_p = _tc_head(
        s0, ss[0], ss[1], ss[2], ss[3], pw0p, pb0, pred_w, pred_b,
        mlp_w1, mlp_b1.reshape(1, OUT), mlp_w2, mlp_b2.reshape(1, BOT),
        mean_w, mean_b.reshape(1, LAT), std_w, std_b.reshape(1, LAT))
    return (mean_p[:BG], std_p[:BG])


# speedup vs baseline: 4.1174x; 1.2667x over previous
import functools

import jax
import jax.numpy as jnp
from jax import lax
from jax.experimental import pallas as pl
from jax.experimental.pallas import tpu as pltpu
from jax.experimental.pallas import tpu_sc as plsc

N = 50000
E = 800000
H = 128
OUT = 256
BOT = 128
LAT = 64
BG = 50
BGP = 64

N_PAD = 50176
N_ACC = 50304
E_PAD = 819200
KI = 4
BLK = 512


_SC_SCRATCH_IDX = 4


def _sc_scratch(F):
    return [
        pltpu.VMEM((KI, 128), jnp.int32),
        pltpu.VMEM((KI, 128), jnp.int32),
        pltpu.VMEM((KI, 128), jnp.int32),
        pltpu.VMEM((KI, 128), jnp.int32),
        pltpu.VMEM((KI * 128, F), jnp.float32),
        pltpu.VMEM_SHARED((N_ACC, F), jnp.float32),
        pltpu.SemaphoreType.DMA,
        pltpu.SemaphoreType.DMA,
        pltpu.SemaphoreType.DMA,
    ]


def _run_edges_pipelined(h_hbm, src_hbm, dst_hbm, acc, rows,
                         ia_s, ia_d, ib_s, ib_d, sem_g, sem_s, sem_i,
                         row0, n_macros):
    pltpu.sync_copy(src_hbm.at[pl.ds(row0, KI)], ia_s)
    pltpu.sync_copy(dst_hbm.at[pl.ds(row0, KI)], ia_d)

    def half(is_, id_, os_, od_, nxt_idx):
        gs = [pltpu.async_copy(h_hbm.at[is_.at[j]],
                               rows.at[pl.ds(j * 128, 128)], sem_g)
              for j in range(KI)]
        ps = pltpu.async_copy(src_hbm.at[pl.ds(nxt_idx, KI)], os_, sem_i)
        pd = pltpu.async_copy(dst_hbm.at[pl.ds(nxt_idx, KI)], od_, sem_i)
        ss = []
        for j in range(KI):
            gs[j].wait()
            ss.append(pltpu.async_copy(rows.at[pl.ds(j * 128, 128)],
                                       acc.at[id_.at[j]], sem_s, add=True))
        ps.wait()
        pd.wait()
        for s_ in ss:
            s_.wait()

    @pl.loop(0, n_macros // 2)
    def _(t):
        a = row0 + (2 * t) * KI
        nxt = row0 + lax.rem(2 * t + 2, n_macros) * KI
        half(ia_s, ia_d, ib_s, ib_d, a + KI)
        half(ib_s, ib_d, ia_s, ia_d, nxt)


def _sc_segsum_pair(x16, src2d, dst2d, zeros16):
    mesh = plsc.VectorSubcoreMesh(core_axis_name="c", subcore_axis_name="s")
    F = x16.shape[1]
    rows_per_worker = E_PAD // 32 // 128
    n_macros = rows_per_worker // KI
    zrows = N_ACC // 16
    orows = N_PAD // 16

    @functools.partial(
        pl.kernel,
        out_type=jax.ShapeDtypeStruct((2, N_PAD, F), jnp.float32),
        mesh=mesh,
        compiler_params=pltpu.CompilerParams(use_tc_tiling_on_sc=False),
        scratch_types=_sc_scratch(F),
    )
    def k(x_hbm, src_hbm, dst_hbm, z_hbm, out_hbm,
          ia_s, ia_d, ib_s, ib_d, rows, acc, sem_g, sem_s, sem_i):
        cid = lax.axis_index("c")
        sid = lax.axis_index("s")
        wid = sid * 2 + cid

        pltpu.sync_copy(z_hbm.at[pl.ds(sid * zrows, zrows)],
                        acc.at[pl.ds(sid * zrows, zrows)])
        plsc.subcore_barrier()

        _run_edges_pipelined(x_hbm, src_hbm, dst_hbm, acc, rows,
                             ia_s, ia_d, ib_s, ib_d, sem_g, sem_s, sem_i,
                             wid * rows_per_worker, n_macros)

        plsc.subcore_barrier()
        pltpu.sync_copy(acc.at[pl.ds(sid * orows, orows)],
                        out_hbm.at[cid].at[pl.ds(sid * orows, orows)])

    return k(x16, src2d, dst2d, zeros16)


def _sc_segsum_one_round(hA, hB, src2d, dst2d, zeros32):
    mesh = plsc.VectorSubcoreMesh(core_axis_name="c", subcore_axis_name="s")
    F = hA.shape[1]
    rows_per_sub = E_PAD // 16 // 128
    n_macros = rows_per_sub // KI
    zrows = N_ACC // 16
    orows = N_PAD // 16
    ot = jax.ShapeDtypeStruct((N_PAD, F), jnp.float32)

    @functools.partial(
        pl.kernel,
        out_type=(ot, ot),
        mesh=mesh,
        compiler_params=pltpu.CompilerParams(use_tc_tiling_on_sc=False),
        scratch_types=_sc_scratch(F),
    )
    def k(hA_hbm, hB_hbm, src_hbm, dst_hbm, z_hbm, oA_hbm, oB_hbm,
          ia_s, ia_d, ib_s, ib_d, rows, acc, sem_g, sem_s, sem_i):
        cid = lax.axis_index("c")
        sid = lax.axis_index("s")
        row0 = sid * rows_per_sub

        def do_chunk(h_hbm, o_hbm):
            pltpu.sync_copy(z_hbm.at[pl.ds(sid * zrows, zrows)],
                            acc.at[pl.ds(sid * zrows, zrows)])
            plsc.subcore_barrier()

            _run_edges_pipelined(h_hbm, src_hbm, dst_hbm, acc, rows,
                                 ia_s, ia_d, ib_s, ib_d, sem_g, sem_s, sem_i,
                                 row0, n_macros)

            plsc.subcore_barrier()
            pltpu.sync_copy(acc.at[pl.ds(sid * orows, orows)],
                            o_hbm.at[pl.ds(sid * orows, orows)])

        @pl.when(cid == 0)
        def _():
            do_chunk(hA_hbm, oA_hbm)

        @pl.when(cid == 1)
        def _():
            do_chunk(hB_hbm, oB_hbm)

    return k(hA, hB, src2d, dst2d, zeros32)


def _iota_cols(shape):
    return lax.broadcasted_iota(jnp.int32, shape, dimension=1)


def _tc_layer0(p0, x16, gids2d, w1p, b1, w2, b2):
    grid = N_PAD // BLK
    oc = jax.ShapeDtypeStruct((N_PAD, 32), jnp.float32)

    def body(p_ref, x_ref, g_ref, w1_ref, b1_ref, w2_ref, b2_ref,
             o0, o1, o2, o3, s0_ref, s1_ref):
        i = pl.program_id(0)
        a = p_ref[0] + p_ref[1] + x_ref[...]
        t = jnp.maximum(jnp.dot(a, w1_ref[...],
                                preferred_element_type=jnp.float32)
                        + b1_ref[...], 0.0)
        h = jnp.maximum(jnp.dot(t, w2_ref[...],
                                preferred_element_type=jnp.float32)
                        + b2_ref[...], 0.0)
        outs = (o0, o1, o2, o3)
        for c in range(4):
            outs[c][...] = h[:, c * 32:(c + 1) * 32]
        ind = (g_ref[...] == _iota_cols((BLK, BGP))).astype(jnp.float32)
        c0 = lax.dot_general(ind, x_ref[...], (((0,), (0,)), ((), ())),
                             preferred_element_type=jnp.float32)
        c1 = lax.dot_general(ind, h, (((0,), (0,)), ((), ())),
                             preferred_element_type=jnp.float32)

        @pl.when(i == 0)
        def _():
            s0_ref[...] = c0
            s1_ref[...] = c1

        @pl.when(i > 0)
        def _():
            s0_ref[...] += c0
            s1_ref[...] += c1

    return pl.pallas_call(
        body,
        grid=(grid,),
        in_specs=[
            pl.BlockSpec((2, BLK, 16), lambda i: (0, i, 0)),
            pl.BlockSpec((BLK, 16), lambda i: (i, 0)),
            pl.BlockSpec((BLK, 1), lambda i: (i, 0)),
            pl.BlockSpec((16, H), lambda i: (0, 0)),
            pl.BlockSpec((1, H), lambda i: (0, 0)),
            pl.BlockSpec((H, H), lambda i: (0, 0)),
            pl.BlockSpec((1, H), lambda i: (0, 0)),
        ],
        out_specs=[
            pl.BlockSpec((BLK, 32), lambda i: (i, 0)),
            pl.BlockSpec((BLK, 32), lambda i: (i, 0)),
            pl.BlockSpec((BLK, 32), lambda i: (i, 0)),
            pl.BlockSpec((BLK, 32), lambda i: (i, 0)),
            pl.BlockSpec((BGP, 16), lambda i: (0, 0)),
            pl.BlockSpec((BGP, H), lambda i: (0, 0)),
        ],
        out_shape=(oc, oc, oc, oc,
                   jax.ShapeDtypeStruct((BGP, 16), jnp.float32),
                   jax.ShapeDtypeStruct((BGP, H), jnp.float32)),
    )(p0, x16, gids2d, w1p, b1, w2, b2)


def _tc_layer(aggs, hs, gids2d, w1, b1, w2, b2, emit_h):
    grid = N_PAD // BLK
    oc = jax.ShapeDtypeStruct((N_PAD, 32), jnp.float32)

    def body(*refs):
        (a0, a1, a2, a3, h0, h1, h2, h3, g_ref, w1_ref, b1_ref, w2_ref,
         b2_ref) = refs[:13]
        outs = refs[13:]
        i = pl.program_id(0)
        arefs = (a0, a1, a2, a3)
        hrefs = (h0, h1, h2, h3)
        a = jnp.concatenate(
            [arefs[c][...] + hrefs[c][...] for c in range(4)], axis=1)
        t = jnp.maximum(jnp.dot(a, w1_ref[...],
                                preferred_element_type=jnp.float32)
                        + b1_ref[...], 0.0)
        h = jnp.maximum(jnp.dot(t, w2_ref[...],
                                preferred_element_type=jnp.float32)
                        + b2_ref[...], 0.0)
        if emit_h:
            for c in range(4):
                outs[c][...] = h[:, c * 32:(c + 1) * 32]
        s_ref = outs[-1]
        ind = (g_ref[...] == _iota_cols((BLK, BGP))).astype(jnp.float32)
        contrib = lax.dot_general(ind, h, (((0,), (0,)), ((), ())),
                                  preferred_element_type=jnp.float32)

        @pl.when(i == 0)
        def _():
            s_ref[...] = contrib

        @pl.when(i > 0)
        def _():
            s_ref[...] += contrib

    chunk_spec = pl.BlockSpec((BLK, 32), lambda i: (i, 0))
    in_specs = ([chunk_spec] * 4 + [chunk_spec] * 4 +
                [pl.BlockSpec((BLK, 1), lambda i: (i, 0)),
                 pl.BlockSpec((H, H), lambda i: (0, 0)),
                 pl.BlockSpec((1, H), lambda i: (0, 0)),
                 pl.BlockSpec((H, H), lambda i: (0, 0)),
                 pl.BlockSpec((1, H), lambda i: (0, 0))])
    s_shape = jax.ShapeDtypeStruct((BGP, H), jnp.float32)
    s_spec = pl.BlockSpec((BGP, H), lambda i: (0, 0))
    if emit_h:
        out_specs = [chunk_spec] * 4 + [s_spec]
        out_shape = (oc, oc, oc, oc, s_shape)
    else:
        out_specs = [s_spec]
        out_shape = (s_shape,)

    return pl.pallas_call(
        body,
        grid=(grid,),
        in_specs=in_specs,
        out_specs=out_specs,
        out_shape=out_shape,
    )(*aggs, *hs, gids2d, w1, b1, w2, b2)


def _tc_head(s0, s1, s2, s3, s4, pw0p, pb0, pred_w, pred_b,
             mlp_w1, mlp_b1, mlp_w2, mlp_b2, mean_w, mean_b, std_w, std_b):
    def body(s0_ref, s1_ref, s2_ref, s3_ref, s4_ref, pw0_ref, pb0_ref,
             pw_ref, pb_ref, w1_ref, b1_ref, w2_ref, b2_ref,
             mw_ref, mb_ref, sw_ref, sb_ref, mean_ref, std_ref):
        dot = functools.partial(jnp.dot, preferred_element_type=jnp.float32)
        score = dot(s0_ref[...], pw0_ref[...]) + pb0_ref[...]
        srefs = (s1_ref, s2_ref, s3_ref, s4_ref)
        for i in range(4):
            score = score + dot(srefs[i][...], pw_ref[i])
        score = score + jnp.sum(pb_ref[...], axis=0, keepdims=True)
        feat = jnp.maximum(dot(score, w1_ref[...]) + b1_ref[...], 0.0)
        feat = jnp.maximum(dot(feat, w2_ref[...]) + b2_ref[...], 0.0)
        mean_ref[...] = dot(feat, mw_ref[...]) + mb_ref[...]
        z = dot(feat, sw_ref[...]) + sb_ref[...]
        std_ref[...] = jnp.maximum(z, 0.0) + jnp.log1p(jnp.exp(-jnp.abs(z)))

    full = lambda s: pl.BlockSpec(s, lambda: tuple(0 for _ in s))
    args = (s0, s1, s2, s3, s4, pw0p, pb0, pred_w, pred_b, mlp_w1, mlp_b1,
            mlp_w2, mlp_b2, mean_w, mean_b, std_w, std_b)
    return pl.pallas_call(
        body,
        in_specs=[full(a.shape) for a in args],
        out_specs=[full((BGP, LAT)), full((BGP, LAT))],
        out_shape=(jax.ShapeDtypeStruct((BGP, LAT), jnp.float32),
                   jax.ShapeDtypeStruct((BGP, LAT), jnp.float32)),
    )(*args)


def kernel(x, edge_index, graph_ids, gin_w1_0, gin_b1_0, gin_w2_0, gin_b2_0,
           gin_w1, gin_b1, gin_w2, gin_b2, pred_w0, pred_b0, pred_w, pred_b,
           mlp_w1, mlp_b1, mlp_w2, mlp_b2, mean_w, mean_b, std_w, std_b):
    src = edge_index[0]
    dst = edge_index[1]
    pad_e = E_PAD - E
    src2d = jnp.concatenate(
        [src, jnp.zeros((pad_e,), jnp.int32)]).reshape(E_PAD // 128, 128)
    dst2d = jnp.concatenate(
        [dst, jnp.full((pad_e,), N_PAD, jnp.int32)]).reshape(E_PAD // 128, 128)

    x16 = jnp.pad(x, ((0, N_PAD - N), (0, 12)))
    gids2d = jnp.pad(graph_ids, (0, N_PAD - N),
                     constant_values=BGP - 1).reshape(N_PAD, 1)
    zeros16 = jnp.zeros((N_ACC, 16), jnp.float32)
    zeros32 = jnp.zeros((N_ACC, 32), jnp.float32)

    w1_0p = jnp.pad(gin_w1_0, ((0, 12), (0, 0)))
    pw0p = jnp.pad(pred_w0, ((0, 12), (0, 0)))
    b1_0 = gin_b1_0.reshape(1, H)
    b2_0 = gin_b2_0.reshape(1, H)
    pb0 = pred_b0.reshape(1, OUT)

    def sc_big(h0, h1, h2, h3):
        a0, a2 = _sc_segsum_one_round(h0, h2, src2d, dst2d, zeros32)
        a1, a3 = _sc_segsum_one_round(h1, h3, src2d, dst2d, zeros32)
        return (a0, a1, a2, a3)

    p0 = _sc_segsum_pair(x16, src2d, dst2d, zeros16)
    h1c = _tc_layer0(p0, x16, gids2d, w1_0p, b1_0, gin_w2_0, b2_0)
    hc = h1c[:4]
    s0, s1 = h1c[4], h1c[5]

    ss = [s1]
    for i in range(3):
        agg = sc_big(*hc)
        res = _tc_layer(agg, hc, gids2d, gin_w1[i], gin_b1[i].reshape(1, H),
                        gin_w2[i], gin_b2[i].reshape(1, H), emit_h=(i < 2))
        if i < 2:
            hc = res[:4]
            ss.append(res[4])
        else:
            ss.append(res[0])

    mean_p, std_p = _tc_head(
        s0, ss[0], ss[1], ss[2], ss[3], pw0p, pb0, pred_w, pred_b,
        mlp_w1, mlp_b1.reshape(1, OUT), mlp_w2, mlp_b2.reshape(1, BOT),
        mean_w, mean_b.reshape(1, LAT), std_w, std_b.reshape(1, LAT))
    return (mean_p[:BG], std_p[:BG])


# speedup vs baseline: 5.9092x; 1.4352x over previous
import functools

import jax
import jax.numpy as jnp
from jax import lax
from jax.experimental import pallas as pl
from jax.experimental.pallas import tpu as pltpu
from jax.experimental.pallas import tpu_sc as plsc

N = 50000
E = 800000
H = 128
OUT = 256
BOT = 128
LAT = 64
BG = 50
BGP = 64

N_PAD = 50176
N_ACC = 50304
E_PAD = 819200
KI = 4
BLK = 512


def _sc_scratch(F, ki, dtype):
    return [
        pltpu.VMEM((ki, 128), jnp.int32),
        pltpu.VMEM((ki, 128), jnp.int32),
        pltpu.VMEM((ki, 128), jnp.int32),
        pltpu.VMEM((ki, 128), jnp.int32),
        pltpu.VMEM((ki * 128, F), dtype),
        pltpu.VMEM_SHARED((N_ACC, F), dtype),
        pltpu.SemaphoreType.DMA,
        pltpu.SemaphoreType.DMA,
        pltpu.SemaphoreType.DMA,
    ]


def _run_edges_pipelined(h_hbm, src_hbm, dst_hbm, acc, rows,
                         ia_s, ia_d, ib_s, ib_d, sem_g, sem_s, sem_i,
                         row0, n_macros, ki):
    pltpu.sync_copy(src_hbm.at[pl.ds(row0, ki)], ia_s)
    pltpu.sync_copy(dst_hbm.at[pl.ds(row0, ki)], ia_d)

    def half(is_, id_, os_, od_, nxt_idx):
        gs = [pltpu.async_copy(h_hbm.at[is_.at[j]],
                               rows.at[pl.ds(j * 128, 128)], sem_g)
              for j in range(ki)]
        ps = pltpu.async_copy(src_hbm.at[pl.ds(nxt_idx, ki)], os_, sem_i)
        pd = pltpu.async_copy(dst_hbm.at[pl.ds(nxt_idx, ki)], od_, sem_i)
        ss = []
        for j in range(ki):
            gs[j].wait()
            ss.append(pltpu.async_copy(rows.at[pl.ds(j * 128, 128)],
                                       acc.at[id_.at[j]], sem_s, add=True))
        ps.wait()
        pd.wait()
        for s_ in ss:
            s_.wait()

    @pl.loop(0, n_macros // 2)
    def _(t):
        a = row0 + (2 * t) * ki
        nxt = row0 + lax.rem(2 * t + 2, n_macros) * ki
        half(ia_s, ia_d, ib_s, ib_d, a + ki)
        half(ib_s, ib_d, ia_s, ia_d, nxt)


def _sc_segsum_pair(x16, src2d, dst2d, zeros16):
    mesh = plsc.VectorSubcoreMesh(core_axis_name="c", subcore_axis_name="s")
    F = x16.shape[1]
    rows_per_worker = E_PAD // 32 // 128
    ki = 4
    n_macros = rows_per_worker // ki
    zrows = N_ACC // 16
    orows = N_PAD // 16

    @functools.partial(
        pl.kernel,
        out_type=jax.ShapeDtypeStruct((2, N_PAD, F), jnp.float32),
        mesh=mesh,
        compiler_params=pltpu.CompilerParams(use_tc_tiling_on_sc=False),
        scratch_types=_sc_scratch(F, ki, jnp.float32),
    )
    def k(x_hbm, src_hbm, dst_hbm, z_hbm, out_hbm,
          ia_s, ia_d, ib_s, ib_d, rows, acc, sem_g, sem_s, sem_i):
        cid = lax.axis_index("c")
        sid = lax.axis_index("s")
        wid = sid * 2 + cid

        pltpu.sync_copy(z_hbm.at[pl.ds(sid * zrows, zrows)],
                        acc.at[pl.ds(sid * zrows, zrows)])
        plsc.subcore_barrier()

        _run_edges_pipelined(x_hbm, src_hbm, dst_hbm, acc, rows,
                             ia_s, ia_d, ib_s, ib_d, sem_g, sem_s, sem_i,
                             wid * rows_per_worker, n_macros, ki)

        plsc.subcore_barrier()
        pltpu.sync_copy(acc.at[pl.ds(sid * orows, orows)],
                        out_hbm.at[cid].at[pl.ds(sid * orows, orows)])

    return k(x16, src2d, dst2d, zeros16)


def _sc_segsum_one_round(hA, hB, src2d, dst2d, zeros32):
    mesh = plsc.VectorSubcoreMesh(core_axis_name="c", subcore_axis_name="s")
    F = hA.shape[1]
    rows_per_sub = E_PAD // 16 // 128
    ki = 8
    n_macros = rows_per_sub // ki
    zrows = N_ACC // 16
    orows = N_PAD // 16
    ot = jax.ShapeDtypeStruct((N_PAD, F), jnp.bfloat16)

    @functools.partial(
        pl.kernel,
        out_type=(ot, ot),
        mesh=mesh,
        compiler_params=pltpu.CompilerParams(use_tc_tiling_on_sc=False),
        scratch_types=_sc_scratch(F, ki, jnp.bfloat16),
    )
    def k(hA_hbm, hB_hbm, src_hbm, dst_hbm, z_hbm, oA_hbm, oB_hbm,
          ia_s, ia_d, ib_s, ib_d, rows, acc, sem_g, sem_s, sem_i):
        cid = lax.axis_index("c")
        sid = lax.axis_index("s")
        row0 = sid * rows_per_sub

        def do_chunk(h_hbm, o_hbm):
            pltpu.sync_copy(z_hbm.at[pl.ds(sid * zrows, zrows)],
                            acc.at[pl.ds(sid * zrows, zrows)])
            plsc.subcore_barrier()

            _run_edges_pipelined(h_hbm, src_hbm, dst_hbm, acc, rows,
                                 ia_s, ia_d, ib_s, ib_d, sem_g, sem_s, sem_i,
                                 row0, n_macros, ki)

            plsc.subcore_barrier()
            pltpu.sync_copy(acc.at[pl.ds(sid * orows, orows)],
                            o_hbm.at[pl.ds(sid * orows, orows)])

        @pl.when(cid == 0)
        def _():
            do_chunk(hA_hbm, oA_hbm)

        @pl.when(cid == 1)
        def _():
            do_chunk(hB_hbm, oB_hbm)

    return k(hA, hB, src2d, dst2d, zeros32)


def _iota_cols(shape):
    return lax.broadcasted_iota(jnp.int32, shape, dimension=1)


def _tc_layer0(p0, x16, gids2d, w1p, b1, w2, b2):
    grid = N_PAD // BLK
    oc = jax.ShapeDtypeStruct((N_PAD, 32), jnp.bfloat16)

    def body(p_ref, x_ref, g_ref, w1_ref, b1_ref, w2_ref, b2_ref,
             o0, o1, o2, o3, s0_ref, s1_ref):
        i = pl.program_id(0)
        a = p_ref[0] + p_ref[1] + x_ref[...]
        t = jnp.maximum(jnp.dot(a, w1_ref[...],
                                preferred_element_type=jnp.float32)
                        + b1_ref[...], 0.0)
        h = jnp.maximum(jnp.dot(t, w2_ref[...],
                                preferred_element_type=jnp.float32)
                        + b2_ref[...], 0.0)
        outs = (o0, o1, o2, o3)
        for c in range(4):
            outs[c][...] = h[:, c * 32:(c + 1) * 32].astype(jnp.bfloat16)
        ind = (g_ref[...] == _iota_cols((BLK, BGP))).astype(jnp.float32)
        c0 = lax.dot_general(ind, x_ref[...], (((0,), (0,)), ((), ())),
                             preferred_element_type=jnp.float32)
        c1 = lax.dot_general(ind, h, (((0,), (0,)), ((), ())),
                             preferred_element_type=jnp.float32)

        @pl.when(i == 0)
        def _():
            s0_ref[...] = c0
            s1_ref[...] = c1

        @pl.when(i > 0)
        def _():
            s0_ref[...] += c0
            s1_ref[...] += c1

    return pl.pallas_call(
        body,
        grid=(grid,),
        in_specs=[
            pl.BlockSpec((2, BLK, 16), lambda i: (0, i, 0)),
            pl.BlockSpec((BLK, 16), lambda i: (i, 0)),
            pl.BlockSpec((BLK, 1), lambda i: (i, 0)),
            pl.BlockSpec((16, H), lambda i: (0, 0)),
            pl.BlockSpec((1, H), lambda i: (0, 0)),
            pl.BlockSpec((H, H), lambda i: (0, 0)),
            pl.BlockSpec((1, H), lambda i: (0, 0)),
        ],
        out_specs=[
            pl.BlockSpec((BLK, 32), lambda i: (i, 0)),
            pl.BlockSpec((BLK, 32), lambda i: (i, 0)),
            pl.BlockSpec((BLK, 32), lambda i: (i, 0)),
            pl.BlockSpec((BLK, 32), lambda i: (i, 0)),
            pl.BlockSpec((BGP, 16), lambda i: (0, 0)),
            pl.BlockSpec((BGP, H), lambda i: (0, 0)),
        ],
        out_shape=(oc, oc, oc, oc,
                   jax.ShapeDtypeStruct((BGP, 16), jnp.float32),
                   jax.ShapeDtypeStruct((BGP, H), jnp.float32)),
    )(p0, x16, gids2d, w1p, b1, w2, b2)


def _tc_layer(aggs, hs, gids2d, w1, b1, w2, b2, emit_h):
    grid = N_PAD // BLK
    oc = jax.ShapeDtypeStruct((N_PAD, 32), jnp.bfloat16)

    def body(*refs):
        (a0, a1, a2, a3, h0, h1, h2, h3, g_ref, w1_ref, b1_ref, w2_ref,
         b2_ref) = refs[:13]
        outs = refs[13:]
        i = pl.program_id(0)
        arefs = (a0, a1, a2, a3)
        hrefs = (h0, h1, h2, h3)
        a = jnp.concatenate(
            [arefs[c][...].astype(jnp.float32) +
             hrefs[c][...].astype(jnp.float32) for c in range(4)], axis=1)
        t = jnp.maximum(jnp.dot(a, w1_ref[...],
                                preferred_element_type=jnp.float32)
                        + b1_ref[...], 0.0)
        h = jnp.maximum(jnp.dot(t, w2_ref[...],
                                preferred_element_type=jnp.float32)
                        + b2_ref[...], 0.0)
        if emit_h:
            for c in range(4):
                outs[c][...] = h[:, c * 32:(c + 1) * 32].astype(jnp.bfloat16)
        s_ref = outs[-1]
        ind = (g_ref[...] == _iota_cols((BLK, BGP))).astype(jnp.float32)
        contrib = lax.dot_general(ind, h, (((0,), (0,)), ((), ())),
                                  preferred_element_type=jnp.float32)

        @pl.when(i == 0)
        def _():
            s_ref[...] = contrib

        @pl.when(i > 0)
        def _():
            s_ref[...] += contrib

    chunk_spec = pl.BlockSpec((BLK, 32), lambda i: (i, 0))
    in_specs = ([chunk_spec] * 4 + [chunk_spec] * 4 +
                [pl.BlockSpec((BLK, 1), lambda i: (i, 0)),
                 pl.BlockSpec((H, H), lambda i: (0, 0)),
                 pl.BlockSpec((1, H), lambda i: (0, 0)),
                 pl.BlockSpec((H, H), lambda i: (0, 0)),
                 pl.BlockSpec((1, H), lambda i: (0, 0))])
    s_shape = jax.ShapeDtypeStruct((BGP, H), jnp.float32)
    s_spec = pl.BlockSpec((BGP, H), lambda i: (0, 0))
    if emit_h:
        out_specs = [chunk_spec] * 4 + [s_spec]
        out_shape = (oc, oc, oc, oc, s_shape)
    else:
        out_specs = [s_spec]
        out_shape = (s_shape,)

    return pl.pallas_call(
        body,
        grid=(grid,),
        in_specs=in_specs,
        out_specs=out_specs,
        out_shape=out_shape,
    )(*aggs, *hs, gids2d, w1, b1, w2, b2)


def _tc_head(s0, s1, s2, s3, s4, pw0p, pb0, pred_w, pred_b,
             mlp_w1, mlp_b1, mlp_w2, mlp_b2, mean_w, mean_b, std_w, std_b):
    def body(s0_ref, s1_ref, s2_ref, s3_ref, s4_ref, pw0_ref, pb0_ref,
             pw_ref, pb_ref, w1_ref, b1_ref, w2_ref, b2_ref,
             mw_ref, mb_ref, sw_ref, sb_ref, mean_ref, std_ref):
        dot = functools.partial(jnp.dot, preferred_element_type=jnp.float32)
        score = dot(s0_ref[...], pw0_ref[...]) + pb0_ref[...]
        srefs = (s1_ref, s2_ref, s3_ref, s4_ref)
        for i in range(4):
            score = score + dot(srefs[i][...], pw_ref[i])
        score = score + jnp.sum(pb_ref[...], axis=0, keepdims=True)
        feat = jnp.maximum(dot(score, w1_ref[...]) + b1_ref[...], 0.0)
        feat = jnp.maximum(dot(feat, w2_ref[...]) + b2_ref[...], 0.0)
        mean_ref[...] = dot(feat, mw_ref[...]) + mb_ref[...]
        z = dot(feat, sw_ref[...]) + sb_ref[...]
        std_ref[...] = jnp.maximum(z, 0.0) + jnp.log1p(jnp.exp(-jnp.abs(z)))

    full = lambda s: pl.BlockSpec(s, lambda: tuple(0 for _ in s))
    args = (s0, s1, s2, s3, s4, pw0p, pb0, pred_w, pred_b, mlp_w1, mlp_b1,
            mlp_w2, mlp_b2, mean_w, mean_b, std_w, std_b)
    return pl.pallas_call(
        body,
        in_specs=[full(a.shape) for a in args],
        out_specs=[full((BGP, LAT)), full((BGP, LAT))],
        out_shape=(jax.ShapeDtypeStruct((BGP, LAT), jnp.float32),
                   jax.ShapeDtypeStruct((BGP, LAT), jnp.float32)),
    )(*args)


def kernel(x, edge_index, graph_ids, gin_w1_0, gin_b1_0, gin_w2_0, gin_b2_0,
           gin_w1, gin_b1, gin_w2, gin_b2, pred_w0, pred_b0, pred_w, pred_b,
           mlp_w1, mlp_b1, mlp_w2, mlp_b2, mean_w, mean_b, std_w, std_b):
    src = edge_index[0]
    dst = edge_index[1]
    pad_e = E_PAD - E
    src2d = jnp.concatenate(
        [src, jnp.zeros((pad_e,), jnp.int32)]).reshape(E_PAD // 128, 128)
    dst2d = jnp.concatenate(
        [dst, jnp.full((pad_e,), N_PAD, jnp.int32)]).reshape(E_PAD // 128, 128)

    x16 = jnp.pad(x, ((0, N_PAD - N), (0, 12)))
    gids2d = jnp.pad(graph_ids, (0, N_PAD - N),
                     constant_values=BGP - 1).reshape(N_PAD, 1)
    zeros16 = jnp.zeros((N_ACC, 16), jnp.float32)
    zeros32 = jnp.zeros((N_ACC, 32), jnp.bfloat16)

    w1_0p = jnp.pad(gin_w1_0, ((0, 12), (0, 0)))
    pw0p = jnp.pad(pred_w0, ((0, 12), (0, 0)))
    b1_0 = gin_b1_0.reshape(1, H)
    b2_0 = gin_b2_0.reshape(1, H)
    pb0 = pred_b0.reshape(1, OUT)

    def sc_big(h0, h1, h2, h3):
        a0, a2 = _sc_segsum_one_round(h0, h2, src2d, dst2d, zeros32)
        a1, a3 = _sc_segsum_one_round(h1, h3, src2d, dst2d, zeros32)
        return (a0, a1, a2, a3)

    p0 = _sc_segsum_pair(x16, src2d, dst2d, zeros16)
    h1c = _tc_layer0(p0, x16, gids2d, w1_0p, b1_0, gin_w2_0, b2_0)
    hc = h1c[:4]
    s0, s1 = h1c[4], h1c[5]

    ss = [s1]
    for i in range(3):
        agg = sc_big(*hc)
        res = _tc_layer(agg, hc, gids2d, gin_w1[i], gin_b1[i].reshape(1, H),
                        gin_w2[i], gin_b2[i].reshape(1, H), emit_h=(i < 2))
        if i < 2:
            hc = res[:4]
            ss.append(res[4])
        else:
            ss.append(res[0])

    mean_p, std_p = _tc_head(
        s0, ss[0], ss[1], ss[2], ss[3], pw0p, pb0, pred_w, pred_b,
        mlp_w1, mlp_b1.reshape(1, OUT), mlp_w2, mlp_b2.reshape(1, BOT),
        mean_w, mean_b.reshape(1, LAT), std_w, std_b.reshape(1, LAT))
    return (mean_p[:BG], std_p[:BG])


# speedup vs baseline: 5.9277x; 1.0031x over previous
import functools

import jax
import jax.numpy as jnp
from jax import lax
from jax.experimental import pallas as pl
from jax.experimental.pallas import tpu as pltpu
from jax.experimental.pallas import tpu_sc as plsc

N = 50000
E = 800000
H = 128
OUT = 256
BOT = 128
LAT = 64
BG = 50
BGP = 64

N_PAD = 50176
N_ACC = 50304
E_PAD = 819200
KI = 4
BLK = 512


def _sc_scratch(F, ki, dtype):
    return [
        pltpu.VMEM((ki, 128), jnp.int32),
        pltpu.VMEM((ki, 128), jnp.int32),
        pltpu.VMEM((ki, 128), jnp.int32),
        pltpu.VMEM((ki, 128), jnp.int32),
        pltpu.VMEM((ki * 128, F), dtype),
        pltpu.VMEM_SHARED((N_ACC, F), dtype),
        pltpu.SemaphoreType.DMA,
        pltpu.SemaphoreType.DMA,
        pltpu.SemaphoreType.DMA,
    ]


def _run_edges_pipelined(h_hbm, src_hbm, dst_hbm, acc, rows,
                         ia_s, ia_d, ib_s, ib_d, sem_g, sem_s, sem_i,
                         row0, n_macros, ki):
    pltpu.sync_copy(src_hbm.at[pl.ds(row0, ki)], ia_s)
    pltpu.sync_copy(dst_hbm.at[pl.ds(row0, ki)], ia_d)

    def half(is_, id_, os_, od_, nxt_idx):
        gs = [pltpu.async_copy(h_hbm.at[is_.at[j]],
                               rows.at[pl.ds(j * 128, 128)], sem_g)
              for j in range(ki)]
        ps = pltpu.async_copy(src_hbm.at[pl.ds(nxt_idx, ki)], os_, sem_i)
        pd = pltpu.async_copy(dst_hbm.at[pl.ds(nxt_idx, ki)], od_, sem_i)
        ss = []
        for j in range(ki):
            gs[j].wait()
            ss.append(pltpu.async_copy(rows.at[pl.ds(j * 128, 128)],
                                       acc.at[id_.at[j]], sem_s, add=True))
        ps.wait()
        pd.wait()
        for s_ in ss:
            s_.wait()

    @pl.loop(0, n_macros // 2)
    def _(t):
        a = row0 + (2 * t) * ki
        nxt = row0 + lax.rem(2 * t + 2, n_macros) * ki
        half(ia_s, ia_d, ib_s, ib_d, a + ki)
        half(ib_s, ib_d, ia_s, ia_d, nxt)


def _sc_segsum_pair(x16, src2d, dst2d, zeros16):
    mesh = plsc.VectorSubcoreMesh(core_axis_name="c", subcore_axis_name="s")
    F = x16.shape[1]
    rows_per_worker = E_PAD // 32 // 128
    ki = 4
    n_macros = rows_per_worker // ki
    zrows = N_ACC // 16
    orows = N_PAD // 16

    @functools.partial(
        pl.kernel,
        out_type=jax.ShapeDtypeStruct((2, N_PAD, F), jnp.float32),
        mesh=mesh,
        compiler_params=pltpu.CompilerParams(use_tc_tiling_on_sc=False),
        scratch_types=_sc_scratch(F, ki, jnp.float32),
    )
    def k(x_hbm, src_hbm, dst_hbm, z_hbm, out_hbm,
          ia_s, ia_d, ib_s, ib_d, rows, acc, sem_g, sem_s, sem_i):
        cid = lax.axis_index("c")
        sid = lax.axis_index("s")
        wid = sid * 2 + cid

        pltpu.sync_copy(z_hbm.at[pl.ds(sid * zrows, zrows)],
                        acc.at[pl.ds(sid * zrows, zrows)])
        plsc.subcore_barrier()

        _run_edges_pipelined(x_hbm, src_hbm, dst_hbm, acc, rows,
                             ia_s, ia_d, ib_s, ib_d, sem_g, sem_s, sem_i,
                             wid * rows_per_worker, n_macros, ki)

        plsc.subcore_barrier()
        pltpu.sync_copy(acc.at[pl.ds(sid * orows, orows)],
                        out_hbm.at[cid].at[pl.ds(sid * orows, orows)])

    return k(x16, src2d, dst2d, zeros16)


def _sc_segsum_one_round(hA, hB, src2d, dst2d, zeros32):
    mesh = plsc.VectorSubcoreMesh(core_axis_name="c", subcore_axis_name="s")
    F = hA.shape[1]
    rows_per_sub = E_PAD // 16 // 128
    ki = 8
    n_macros = rows_per_sub // ki
    zrows = N_ACC // 16
    orows = N_PAD // 16
    ot = jax.ShapeDtypeStruct((N_PAD, F), jnp.bfloat16)

    @functools.partial(
        pl.kernel,
        out_type=(ot, ot),
        mesh=mesh,
        compiler_params=pltpu.CompilerParams(use_tc_tiling_on_sc=False),
        scratch_types=_sc_scratch(F, ki, jnp.bfloat16),
    )
    def k(hA_hbm, hB_hbm, src_hbm, dst_hbm, z_hbm, oA_hbm, oB_hbm,
          ia_s, ia_d, ib_s, ib_d, rows, acc, sem_g, sem_s, sem_i):
        cid = lax.axis_index("c")
        sid = lax.axis_index("s")
        row0 = sid * rows_per_sub

        def do_chunk(h_hbm, o_hbm):
            pltpu.sync_copy(z_hbm.at[pl.ds(sid * zrows, zrows)],
                            acc.at[pl.ds(sid * zrows, zrows)])
            plsc.subcore_barrier()

            _run_edges_pipelined(h_hbm, src_hbm, dst_hbm, acc, rows,
                                 ia_s, ia_d, ib_s, ib_d, sem_g, sem_s, sem_i,
                                 row0, n_macros, ki)

            plsc.subcore_barrier()
            pltpu.sync_copy(acc.at[pl.ds(sid * orows, orows)],
                            o_hbm.at[pl.ds(sid * orows, orows)])

        @pl.when(cid == 0)
        def _():
            do_chunk(hA_hbm, oA_hbm)

        @pl.when(cid == 1)
        def _():
            do_chunk(hB_hbm, oB_hbm)

    return k(hA, hB, src2d, dst2d, zeros32)


def _sc_segsum_dual(h0, h1, h2, h3, src2d, dst2d, zeros32):
    mesh = plsc.VectorSubcoreMesh(core_axis_name="c", subcore_axis_name="s")
    F = 32
    rows_per_sub = E_PAD // 16 // 128
    ki = 4
    n_macros = rows_per_sub // ki
    zrows = N_ACC // 16
    orows = N_PAD // 16
    ot = jax.ShapeDtypeStruct((N_PAD, F), jnp.bfloat16)

    @functools.partial(
        pl.kernel,
        out_type=(ot, ot, ot, ot),
        mesh=mesh,
        compiler_params=pltpu.CompilerParams(use_tc_tiling_on_sc=False),
        scratch_types=[
            pltpu.VMEM((ki, 128), jnp.int32),
            pltpu.VMEM((ki, 128), jnp.int32),
            pltpu.VMEM((ki, 128), jnp.int32),
            pltpu.VMEM((ki, 128), jnp.int32),
            pltpu.VMEM((ki * 128, F), jnp.bfloat16),
            pltpu.VMEM((ki * 128, F), jnp.bfloat16),
            pltpu.VMEM_SHARED((N_ACC, F), jnp.bfloat16),
            pltpu.VMEM_SHARED((N_ACC, F), jnp.bfloat16),
            pltpu.SemaphoreType.DMA,
            pltpu.SemaphoreType.DMA,
            pltpu.SemaphoreType.DMA,
        ],
    )
    def k(h0_hbm, h1_hbm, h2_hbm, h3_hbm, src_hbm, dst_hbm, z_hbm,
          o0_hbm, o1_hbm, o2_hbm, o3_hbm,
          ia_s, ia_d, ib_s, ib_d, rowsA, rowsB, accA, accB,
          sem_g, sem_s, sem_i):
        cid = lax.axis_index("c")
        sid = lax.axis_index("s")
        row0 = sid * rows_per_sub

        def do_pair(hA_hbm, hB_hbm, oA_hbm, oB_hbm):
            pltpu.sync_copy(z_hbm.at[pl.ds(sid * zrows, zrows)],
                            accA.at[pl.ds(sid * zrows, zrows)])
            pltpu.sync_copy(z_hbm.at[pl.ds(sid * zrows, zrows)],
                            accB.at[pl.ds(sid * zrows, zrows)])
            plsc.subcore_barrier()

            pltpu.sync_copy(src_hbm.at[pl.ds(row0, ki)], ia_s)
            pltpu.sync_copy(dst_hbm.at[pl.ds(row0, ki)], ia_d)

            def half(is_, id_, os_, od_, nxt_idx):
                ga = [pltpu.async_copy(hA_hbm.at[is_.at[j]],
                                       rowsA.at[pl.ds(j * 128, 128)], sem_g)
                      for j in range(ki)]
                gb = [pltpu.async_copy(hB_hbm.at[is_.at[j]],
                                       rowsB.at[pl.ds(j * 128, 128)], sem_g)
                      for j in range(ki)]
                ps = pltpu.async_copy(src_hbm.at[pl.ds(nxt_idx, ki)], os_,
                                      sem_i)
                pd = pltpu.async_copy(dst_hbm.at[pl.ds(nxt_idx, ki)], od_,
                                      sem_i)
                ss = []
                for j in range(ki):
                    ga[j].wait()
                    ss.append(pltpu.async_copy(rowsA.at[pl.ds(j * 128, 128)],
                                               accA.at[id_.at[j]], sem_s,
                                               add=True))
                for j in range(ki):
                    gb[j].wait()
                    ss.append(pltpu.async_copy(rowsB.at[pl.ds(j * 128, 128)],
                                               accB.at[id_.at[j]], sem_s,
                                               add=True))
                ps.wait()
                pd.wait()
                for s_ in ss:
                    s_.wait()

            @pl.loop(0, n_macros // 2)
            def _(t):
                nxt = row0 + lax.rem(2 * t + 2, n_macros) * ki
                half(ia_s, ia_d, ib_s, ib_d, row0 + (2 * t) * ki + ki)
                half(ib_s, ib_d, ia_s, ia_d, nxt)

            plsc.subcore_barrier()
            pltpu.sync_copy(accA.at[pl.ds(sid * orows, orows)],
                            oA_hbm.at[pl.ds(sid * orows, orows)])
            pltpu.sync_copy(accB.at[pl.ds(sid * orows, orows)],
                            oB_hbm.at[pl.ds(sid * orows, orows)])

        @pl.when(cid == 0)
        def _():
            do_pair(h0_hbm, h1_hbm, o0_hbm, o1_hbm)

        @pl.when(cid == 1)
        def _():
            do_pair(h2_hbm, h3_hbm, o2_hbm, o3_hbm)

    return k(h0, h1, h2, h3, src2d, dst2d, zeros32)


def _iota_cols(shape):
    return lax.broadcasted_iota(jnp.int32, shape, dimension=1)


def _tc_layer0(p0, x16, gids2d, w1p, b1, w2, b2):
    grid = N_PAD // BLK
    oc = jax.ShapeDtypeStruct((N_PAD, 32), jnp.bfloat16)

    def body(p_ref, x_ref, g_ref, w1_ref, b1_ref, w2_ref, b2_ref,
             o0, o1, o2, o3, s0_ref, s1_ref):
        i = pl.program_id(0)
        a = p_ref[0] + p_ref[1] + x_ref[...]
        t = jnp.maximum(jnp.dot(a, w1_ref[...],
                                preferred_element_type=jnp.float32)
                        + b1_ref[...], 0.0)
        h = jnp.maximum(jnp.dot(t, w2_ref[...],
                                preferred_element_type=jnp.float32)
                        + b2_ref[...], 0.0)
        outs = (o0, o1, o2, o3)
        for c in range(4):
            outs[c][...] = h[:, c * 32:(c + 1) * 32].astype(jnp.bfloat16)
        ind = (g_ref[...] == _iota_cols((BLK, BGP))).astype(jnp.float32)
        c0 = lax.dot_general(ind, x_ref[...], (((0,), (0,)), ((), ())),
                             preferred_element_type=jnp.float32)
        c1 = lax.dot_general(ind, h, (((0,), (0,)), ((), ())),
                             preferred_element_type=jnp.float32)

        @pl.when(i == 0)
        def _():
            s0_ref[...] = c0
            s1_ref[...] = c1

        @pl.when(i > 0)
        def _():
            s0_ref[...] += c0
            s1_ref[...] += c1

    return pl.pallas_call(
        body,
        grid=(grid,),
        in_specs=[
            pl.BlockSpec((2, BLK, 16), lambda i: (0, i, 0)),
            pl.BlockSpec((BLK, 16), lambda i: (i, 0)),
            pl.BlockSpec((BLK, 1), lambda i: (i, 0)),
            pl.BlockSpec((16, H), lambda i: (0, 0)),
            pl.BlockSpec((1, H), lambda i: (0, 0)),
            pl.BlockSpec((H, H), lambda i: (0, 0)),
            pl.BlockSpec((1, H), lambda i: (0, 0)),
        ],
        out_specs=[
            pl.BlockSpec((BLK, 32), lambda i: (i, 0)),
            pl.BlockSpec((BLK, 32), lambda i: (i, 0)),
            pl.BlockSpec((BLK, 32), lambda i: (i, 0)),
            pl.BlockSpec((BLK, 32), lambda i: (i, 0)),
            pl.BlockSpec((BGP, 16), lambda i: (0, 0)),
            pl.BlockSpec((BGP, H), lambda i: (0, 0)),
        ],
        out_shape=(oc, oc, oc, oc,
                   jax.ShapeDtypeStruct((BGP, 16), jnp.float32),
                   jax.ShapeDtypeStruct((BGP, H), jnp.float32)),
    )(p0, x16, gids2d, w1p, b1, w2, b2)


def _tc_layer(aggs, hs, gids2d, w1, b1, w2, b2, emit_h):
    grid = N_PAD // BLK
    oc = jax.ShapeDtypeStruct((N_PAD, 32), jnp.bfloat16)

    def body(*refs):
        (a0, a1, a2, a3, h0, h1, h2, h3, g_ref, w1_ref, b1_ref, w2_ref,
         b2_ref) = refs[:13]
        outs = refs[13:]
        i = pl.program_id(0)
        arefs = (a0, a1, a2, a3)
        hrefs = (h0, h1, h2, h3)
        a = jnp.concatenate(
            [arefs[c][...].astype(jnp.float32) +
             hrefs[c][...].astype(jnp.float32) for c in range(4)], axis=1)
        t = jnp.maximum(jnp.dot(a, w1_ref[...],
                                preferred_element_type=jnp.float32)
                        + b1_ref[...], 0.0)
        h = jnp.maximum(jnp.dot(t, w2_ref[...],
                                preferred_element_type=jnp.float32)
                        + b2_ref[...], 0.0)
        if emit_h:
            for c in range(4):
                outs[c][...] = h[:, c * 32:(c + 1) * 32].astype(jnp.bfloat16)
        s_ref = outs[-1]
        ind = (g_ref[...] == _iota_cols((BLK, BGP))).astype(jnp.float32)
        contrib = lax.dot_general(ind, h, (((0,), (0,)), ((), ())),
                                  preferred_element_type=jnp.float32)

        @pl.when(i == 0)
        def _():
            s_ref[...] = contrib

        @pl.when(i > 0)
        def _():
            s_ref[...] += contrib

    chunk_spec = pl.BlockSpec((BLK, 32), lambda i: (i, 0))
    in_specs = ([chunk_spec] * 4 + [chunk_spec] * 4 +
                [pl.BlockSpec((BLK, 1), lambda i: (i, 0)),
                 pl.BlockSpec((H, H), lambda i: (0, 0)),
                 pl.BlockSpec((1, H), lambda i: (0, 0)),
                 pl.BlockSpec((H, H), lambda i: (0, 0)),
                 pl.BlockSpec((1, H), lambda i: (0, 0))])
    s_shape = jax.ShapeDtypeStruct((BGP, H), jnp.float32)
    s_spec = pl.BlockSpec((BGP, H), lambda i: (0, 0))
    if emit_h:
        out_specs = [chunk_spec] * 4 + [s_spec]
        out_shape = (oc, oc, oc, oc, s_shape)
    else:
        out_specs = [s_spec]
        out_shape = (s_shape,)

    return pl.pallas_call(
        body,
        grid=(grid,),
        in_specs=in_specs,
        out_specs=out_specs,
        out_shape=out_shape,
    )(*aggs, *hs, gids2d, w1, b1, w2, b2)


def _tc_head(s0, s1, s2, s3, s4, pw0p, pb0, pred_w, pred_b,
             mlp_w1, mlp_b1, mlp_w2, mlp_b2, mean_w, mean_b, std_w, std_b):
    def body(s0_ref, s1_ref, s2_ref, s3_ref, s4_ref, pw0_ref, pb0_ref,
             pw_ref, pb_ref, w1_ref, b1_ref, w2_ref, b2_ref,
             mw_ref, mb_ref, sw_ref, sb_ref, mean_ref, std_ref):
        dot = functools.partial(jnp.dot, preferred_element_type=jnp.float32)
        score = dot(s0_ref[...], pw0_ref[...]) + pb0_ref[...]
        srefs = (s1_ref, s2_ref, s3_ref, s4_ref)
        for i in range(4):
            score = score + dot(srefs[i][...], pw_ref[i])
        score = score + jnp.sum(pb_ref[...], axis=0, keepdims=True)
        feat = jnp.maximum(dot(score, w1_ref[...]) + b1_ref[...], 0.0)
        feat = jnp.maximum(dot(feat, w2_ref[...]) + b2_ref[...], 0.0)
        mean_ref[...] = dot(feat, mw_ref[...]) + mb_ref[...]
        z = dot(feat, sw_ref[...]) + sb_ref[...]
        std_ref[...] = jnp.maximum(z, 0.0) + jnp.log1p(jnp.exp(-jnp.abs(z)))

    full = lambda s: pl.BlockSpec(s, lambda: tuple(0 for _ in s))
    args = (s0, s1, s2, s3, s4, pw0p, pb0, pred_w, pred_b, mlp_w1, mlp_b1,
            mlp_w2, mlp_b2, mean_w, mean_b, std_w, std_b)
    return pl.pallas_call(
        body,
        in_specs=[full(a.shape) for a in args],
        out_specs=[full((BGP, LAT)), full((BGP, LAT))],
        out_shape=(jax.ShapeDtypeStruct((BGP, LAT), jnp.float32),
                   jax.ShapeDtypeStruct((BGP, LAT), jnp.float32)),
    )(*args)


def kernel(x, edge_index, graph_ids, gin_w1_0, gin_b1_0, gin_w2_0, gin_b2_0,
           gin_w1, gin_b1, gin_w2, gin_b2, pred_w0, pred_b0, pred_w, pred_b,
           mlp_w1, mlp_b1, mlp_w2, mlp_b2, mean_w, mean_b, std_w, std_b):
    src = edge_index[0]
    dst = edge_index[1]
    pad_e = E_PAD - E
    src2d = jnp.concatenate(
        [src, jnp.zeros((pad_e,), jnp.int32)]).reshape(E_PAD // 128, 128)
    dst2d = jnp.concatenate(
        [dst, jnp.full((pad_e,), N_PAD, jnp.int32)]).reshape(E_PAD // 128, 128)

    x16 = jnp.pad(x, ((0, N_PAD - N), (0, 12)))
    gids2d = jnp.pad(graph_ids, (0, N_PAD - N),
                     constant_values=BGP - 1).reshape(N_PAD, 1)
    zeros16 = jnp.zeros((N_ACC, 16), jnp.float32)
    zeros32 = jnp.zeros((N_ACC, 32), jnp.bfloat16)

    w1_0p = jnp.pad(gin_w1_0, ((0, 12), (0, 0)))
    pw0p = jnp.pad(pred_w0, ((0, 12), (0, 0)))
    b1_0 = gin_b1_0.reshape(1, H)
    b2_0 = gin_b2_0.reshape(1, H)
    pb0 = pred_b0.reshape(1, OUT)

    def sc_big(h0, h1, h2, h3):
        return _sc_segsum_dual(h0, h1, h2, h3, src2d, dst2d, zeros32)

    p0 = _sc_segsum_pair(x16, src2d, dst2d, zeros16)
    h1c = _tc_layer0(p0, x16, gids2d, w1_0p, b1_0, gin_w2_0, b2_0)
    hc = h1c[:4]
    s0, s1 = h1c[4], h1c[5]

    ss = [s1]
    for i in range(3):
        agg = sc_big(*hc)
        res = _tc_layer(agg, hc, gids2d, gin_w1[i], gin_b1[i].reshape(1, H),
                        gin_w2[i], gin_b2[i].reshape(1, H), emit_h=(i < 2))
        if i < 2:
            hc = res[:4]
            ss.append(res[4])
        else:
            ss.append(res[0])

    mean_p, std_p = _tc_head(
        s0, ss[0], ss[1], ss[2], ss[3], pw0p, pb0, pred_w, pred_b,
        mlp_w1, mlp_b1.reshape(1, OUT), mlp_w2, mlp_b2.reshape(1, BOT),
        mean_w, mean_b.reshape(1, LAT), std_w, std_b.reshape(1, LAT))
    return (mean_p[:BG], std_p[:BG])
